# baseline (speedup 1.0000x reference)
import functools

import jax
import jax.numpy as jnp
from jax import lax
from jax.experimental import pallas as pl
from jax.experimental.pallas import tpu as pltpu
from jax.experimental.pallas import tpu_sc as plsc

N_NODES = 10000
D_FEAT = 128
N_EDGES = 320000
NC = 2
NS = 16
CHUNK = 128
CH_PER_TILE = 79
E_TILE = CHUNK * CH_PER_TILE
E_SC = E_TILE * NS
ROWS_PAD = 10240
ROWS_TILE = ROWS_PAD // NS
ZCHUNK = 64
WEIGHT = 0.1


def _sc_agg(xp, src2, dst2):
    mesh = plsc.VectorSubcoreMesh(core_axis_name="c", subcore_axis_name="s")

    @functools.partial(
        pl.kernel,
        out_type=jax.ShapeDtypeStruct((NC, ROWS_PAD, D_FEAT), jnp.float32),
        mesh=mesh,
        scratch_types=[
            pltpu.VMEM((CH_PER_TILE, CHUNK), jnp.int32),
            pltpu.VMEM((CH_PER_TILE, CHUNK), jnp.int32),
            pltpu.VMEM((CHUNK, D_FEAT), jnp.float32),
            pltpu.VMEM((ZCHUNK, D_FEAT), jnp.float32),
            pltpu.VMEM_SHARED((ROWS_PAD, D_FEAT), jnp.float32),
            pltpu.SemaphoreType.DMA,
        ],
        compiler_params=pltpu.CompilerParams(use_tc_tiling_on_sc=False),
    )
    def k(x_hbm, src_hbm, dst_hbm, out_hbm, src_v, dst_v, rows_v, zbuf, acc, sem):
        c = lax.axis_index("c")
        s = lax.axis_index("s")
        row0 = s * ROWS_TILE

        pltpu.sync_copy(src_hbm.at[c, s], src_v)
        pltpu.sync_copy(dst_hbm.at[c, s], dst_v)

        zero = jnp.zeros((16,), jnp.float32)

        def zrow(i, _):
            for j in range(D_FEAT // 16):
                zbuf[i, pl.ds(j * 16, 16)] = zero
            return 0

        lax.fori_loop(0, ZCHUNK, zrow, 0)

        def zchunk(r, _):
            pltpu.sync_copy(zbuf, acc.at[pl.ds(row0 + r * ZCHUNK, ZCHUNK)])
            return 0

        lax.fori_loop(0, ROWS_TILE // ZCHUNK, zchunk, 0)
        plsc.subcore_barrier()

        def edge_chunk(j, _):
            pltpu.async_copy(x_hbm.at[src_v.at[j]], rows_v, sem).wait()
            pltpu.sync_copy(rows_v, acc.at[dst_v.at[j]], add=True)
            return 0

        lax.fori_loop(0, CH_PER_TILE, edge_chunk, 0)
        plsc.subcore_barrier()

        pltpu.sync_copy(acc.at[pl.ds(row0, ROWS_TILE)],
                        out_hbm.at[c, pl.ds(row0, ROWS_TILE)])

    return k(xp, src2, dst2)


def _combine(x, p):
    def body(x_ref, p_ref, o_ref):
        o_ref[...] = x_ref[...] + jnp.float32(WEIGHT) * (p_ref[0] + p_ref[1])

    return pl.pallas_call(
        body,
        out_shape=jax.ShapeDtypeStruct((N_NODES, D_FEAT), jnp.float32),
        grid=(10,),
        in_specs=[
            pl.BlockSpec((1000, D_FEAT), lambda i: (i, 0)),
            pl.BlockSpec((NC, 1000, D_FEAT), lambda i: (0, i, 0)),
        ],
        out_specs=pl.BlockSpec((1000, D_FEAT), lambda i: (i, 0)),
    )(x, p)


@jax.jit
def kernel(x, edge_index):
    src = edge_index[0]
    dst = edge_index[1]
    xp = jnp.pad(x, ((0, ROWS_PAD - N_NODES), (0, 0)))
    half = N_EDGES // NC
    pad = E_SC - half

    def prep(a, fill):
        a0 = jnp.pad(a[:half], (0, pad), constant_values=fill)
        a1 = jnp.pad(a[half:], (0, pad), constant_values=fill)
        return jnp.stack([a0, a1]).reshape(NC, NS, CH_PER_TILE, CHUNK)

    src2 = prep(src, 0)
    dst2 = prep(dst, N_NODES)
    p = _sc_agg(xp, src2, dst2)
    return _combine(x, p)


# speedup vs baseline: 1.1111x; 1.1111x over previous
import functools

import jax
import jax.numpy as jnp
from jax import lax
from jax.experimental import pallas as pl
from jax.experimental.pallas import tpu as pltpu
from jax.experimental.pallas import tpu_sc as plsc

N_NODES = 10000
D_FEAT = 128
HALF = 64
N_EDGES = 320000
NC = 2
NS = 16
CHUNK = 128
CH_PER_TILE = 157
E_TILE = CHUNK * CH_PER_TILE
E_PAD = E_TILE * NS
ROWS_PAD = 10240
ROWS_TILE = ROWS_PAD // NS
ZCHUNK = 64
WEIGHT = 0.1


def _sc_agg(xh, src2, dst2):
    mesh = plsc.VectorSubcoreMesh(core_axis_name="c", subcore_axis_name="s")

    @functools.partial(
        pl.kernel,
        out_type=jax.ShapeDtypeStruct((NC, ROWS_PAD, HALF), jnp.float32),
        mesh=mesh,
        scratch_types=[
            pltpu.VMEM((CH_PER_TILE, CHUNK), jnp.int32),
            pltpu.VMEM((CH_PER_TILE, CHUNK), jnp.int32),
            pltpu.VMEM((CHUNK, HALF), jnp.float32),
            pltpu.VMEM((ZCHUNK, HALF), jnp.float32),
            pltpu.VMEM_SHARED((ROWS_PAD, HALF), jnp.float32),
            pltpu.SemaphoreType.DMA,
        ],
        compiler_params=pltpu.CompilerParams(use_tc_tiling_on_sc=False),
    )
    def k(xh_hbm, src_hbm, dst_hbm, out_hbm, src_v, dst_v, rows_v, zbuf, acc, sem):
        c = lax.axis_index("c")
        s = lax.axis_index("s")
        row0 = s * ROWS_TILE

        pltpu.sync_copy(src_hbm.at[c, s], src_v)
        pltpu.sync_copy(dst_hbm.at[s], dst_v)

        zero = jnp.zeros((16,), jnp.float32)

        def zrow(i, _):
            for j in range(HALF // 16):
                zbuf[i, pl.ds(j * 16, 16)] = zero
            return 0

        lax.fori_loop(0, ZCHUNK, zrow, 0)

        def zchunk(r, _):
            pltpu.sync_copy(zbuf, acc.at[pl.ds(row0 + r * ZCHUNK, ZCHUNK)])
            return 0

        lax.fori_loop(0, ROWS_TILE // ZCHUNK, zchunk, 0)
        plsc.subcore_barrier()

        def edge_chunk(j, _):
            pltpu.async_copy(xh_hbm.at[src_v.at[j]], rows_v, sem).wait()
            pltpu.sync_copy(rows_v, acc.at[dst_v.at[j]], add=True)
            return 0

        lax.fori_loop(0, CH_PER_TILE, edge_chunk, 0)
        plsc.subcore_barrier()

        pltpu.sync_copy(acc.at[pl.ds(row0, ROWS_TILE)],
                        out_hbm.at[c, pl.ds(row0, ROWS_TILE)])

    return k(xh, src2, dst2)


def _combine(x, p):
    def body(x_ref, p_ref, o_ref):
        w = jnp.float32(WEIGHT)
        o_ref[:, :HALF] = x_ref[:, :HALF] + w * p_ref[0]
        o_ref[:, HALF:] = x_ref[:, HALF:] + w * p_ref[1]

    return pl.pallas_call(
        body,
        out_shape=jax.ShapeDtypeStruct((N_NODES, D_FEAT), jnp.float32),
        grid=(10,),
        in_specs=[
            pl.BlockSpec((1000, D_FEAT), lambda i: (i, 0)),
            pl.BlockSpec((NC, 1000, HALF), lambda i: (0, i, 0)),
        ],
        out_specs=pl.BlockSpec((1000, D_FEAT), lambda i: (i, 0)),
    )(x, p)


@jax.jit
def kernel(x, edge_index):
    src = edge_index[0]
    dst = edge_index[1]
    xh = jnp.concatenate([x[:, :HALF], x[:, HALF:]], axis=0)
    srcp = jnp.pad(src, (0, E_PAD - N_EDGES))
    dstp = jnp.pad(dst, (0, E_PAD - N_EDGES), constant_values=N_NODES)
    src2 = jnp.stack([srcp, srcp + N_NODES]).reshape(NC, NS, CH_PER_TILE, CHUNK)
    dst2 = dstp.reshape(NS, CH_PER_TILE, CHUNK)
    p = _sc_agg(xh, src2, dst2)
    return _combine(x, p)


# speedup vs baseline: 1.2763x; 1.1487x over previous
import functools

import jax
import jax.numpy as jnp
from jax import lax
from jax.experimental import pallas as pl
from jax.experimental.pallas import tpu as pltpu
from jax.experimental.pallas import tpu_sc as plsc

N_NODES = 10000
D_FEAT = 128
HALF = 64
N_EDGES = 320000
NC = 2
NS = 16
CHUNK = 128
CH_PER_TILE = 157
E_TILE = CHUNK * CH_PER_TILE
E_PAD = E_TILE * NS
ROWS_PAD = 10240
ROWS_TILE = ROWS_PAD // NS
X_TILE = N_NODES // NS
ZCHUNK = 16
WEIGHT = 0.1


def _sc_agg(xh, src2, dst2):
    mesh = plsc.VectorSubcoreMesh(core_axis_name="c", subcore_axis_name="s")

    @functools.partial(
        pl.kernel,
        out_type=jax.ShapeDtypeStruct((NC, ROWS_PAD, HALF), jnp.float32),
        mesh=mesh,
        scratch_types=[
            pltpu.VMEM((CH_PER_TILE, CHUNK), jnp.int32),
            pltpu.VMEM((CH_PER_TILE, CHUNK), jnp.int32),
            pltpu.VMEM((CHUNK, HALF), jnp.float32),
            pltpu.VMEM((ZCHUNK, HALF), jnp.float32),
            pltpu.VMEM_SHARED((N_NODES, HALF), jnp.float32),
            pltpu.VMEM_SHARED((ROWS_PAD, HALF), jnp.float32),
            pltpu.SemaphoreType.DMA,
        ],
        compiler_params=pltpu.CompilerParams(use_tc_tiling_on_sc=False),
    )
    def k(xh_hbm, src_hbm, dst_hbm, out_hbm, src_v, dst_v, rows_v, zbuf, xs,
          acc, sem):
        c = lax.axis_index("c")
        s = lax.axis_index("s")
        row0 = s * ROWS_TILE

        pltpu.sync_copy(src_hbm.at[s], src_v)
        pltpu.sync_copy(dst_hbm.at[s], dst_v)
        x0 = s * X_TILE
        pltpu.sync_copy(xh_hbm.at[pl.ds(c * N_NODES + x0, X_TILE)],
                        xs.at[pl.ds(x0, X_TILE)])

        zero = jnp.zeros((16,), jnp.float32)

        def zrow(i, _):
            for j in range(HALF // 16):
                zbuf[i, pl.ds(j * 16, 16)] = zero
            return 0

        lax.fori_loop(0, ZCHUNK, zrow, 0)

        def zchunk(r, _):
            pltpu.sync_copy(zbuf, acc.at[pl.ds(row0 + r * ZCHUNK, ZCHUNK)])
            return 0

        lax.fori_loop(0, ROWS_TILE // ZCHUNK, zchunk, 0)
        plsc.subcore_barrier()

        def edge_chunk(j, _):
            pltpu.async_copy(xs.at[src_v.at[j]], rows_v, sem).wait()
            pltpu.sync_copy(rows_v, acc.at[dst_v.at[j]], add=True)
            return 0

        lax.fori_loop(0, CH_PER_TILE, edge_chunk, 0)
        plsc.subcore_barrier()

        pltpu.sync_copy(acc.at[pl.ds(row0, ROWS_TILE)],
                        out_hbm.at[c, pl.ds(row0, ROWS_TILE)])

    return k(xh, src2, dst2)


def _combine(x, p):
    def body(x_ref, p_ref, o_ref):
        w = jnp.float32(WEIGHT)
        o_ref[:, :HALF] = x_ref[:, :HALF] + w * p_ref[0]
        o_ref[:, HALF:] = x_ref[:, HALF:] + w * p_ref[1]

    return pl.pallas_call(
        body,
        out_shape=jax.ShapeDtypeStruct((N_NODES, D_FEAT), jnp.float32),
        grid=(10,),
        in_specs=[
            pl.BlockSpec((1000, D_FEAT), lambda i: (i, 0)),
            pl.BlockSpec((NC, 1000, HALF), lambda i: (0, i, 0)),
        ],
        out_specs=pl.BlockSpec((1000, D_FEAT), lambda i: (i, 0)),
    )(x, p)


@jax.jit
def kernel(x, edge_index):
    src = edge_index[0]
    dst = edge_index[1]
    xh = jnp.concatenate([x[:, :HALF], x[:, HALF:]], axis=0)
    srcp = jnp.pad(src, (0, E_PAD - N_EDGES))
    dstp = jnp.pad(dst, (0, E_PAD - N_EDGES), constant_values=N_NODES)
    src2 = srcp.reshape(NS, CH_PER_TILE, CHUNK)
    dst2 = dstp.reshape(NS, CH_PER_TILE, CHUNK)
    p = _sc_agg(xh, src2, dst2)
    return _combine(x, p)


# speedup vs baseline: 1.6326x; 1.2791x over previous
import functools

import jax
import jax.numpy as jnp
from jax import lax
from jax.experimental import pallas as pl
from jax.experimental.pallas import tpu as pltpu
from jax.experimental.pallas import tpu_sc as plsc

N_NODES = 10000
D_FEAT = 128
HALF = 64
N_EDGES = 320000
NC = 2
NS = 16
CHUNK = 128
NH = 2
CH_PER_HALF = 80
CH_PER_TILE = NH * CH_PER_HALF
E_TILE = CHUNK * CH_PER_TILE
E_PAD = E_TILE * NS
ROWS_PAD = 10240
ROWS_TILE = ROWS_PAD // NS
X_TILE = N_NODES // NS
ZCHUNK = 16
WEIGHT = 0.1


def _sc_agg(xh, src2, dst2):
    mesh = plsc.VectorSubcoreMesh(core_axis_name="c", subcore_axis_name="s")

    @functools.partial(
        pl.kernel,
        out_type=jax.ShapeDtypeStruct((NC, ROWS_PAD, HALF), jnp.float32),
        mesh=mesh,
        scratch_types=[
            pltpu.VMEM((CH_PER_HALF, CHUNK), jnp.int32),
            pltpu.VMEM((CH_PER_HALF, CHUNK), jnp.int32),
            pltpu.VMEM((CHUNK, HALF), jnp.float32),
            pltpu.VMEM((CHUNK, HALF), jnp.float32),
            pltpu.VMEM((ZCHUNK, HALF), jnp.float32),
            pltpu.VMEM_SHARED((N_NODES, HALF), jnp.float32),
            pltpu.VMEM_SHARED((ROWS_PAD, HALF), jnp.float32),
            pltpu.SemaphoreType.DMA,
            pltpu.SemaphoreType.DMA,
            pltpu.SemaphoreType.DMA,
            pltpu.SemaphoreType.DMA,
        ],
        compiler_params=pltpu.CompilerParams(use_tc_tiling_on_sc=False),
    )
    def k(xh_hbm, src_hbm, dst_hbm, out_hbm, src_v, dst_v, rows_a, rows_b,
          zbuf, xs, acc, sem_ga, sem_gb, sem_sa, sem_sb):
        c = lax.axis_index("c")
        s = lax.axis_index("s")
        row0 = s * ROWS_TILE

        x0 = s * X_TILE
        pltpu.sync_copy(xh_hbm.at[pl.ds(c * N_NODES + x0, X_TILE)],
                        xs.at[pl.ds(x0, X_TILE)])

        zero = jnp.zeros((16,), jnp.float32)

        def zrow(i, _):
            for j in range(HALF // 16):
                zbuf[i, pl.ds(j * 16, 16)] = zero
            return 0

        lax.fori_loop(0, ZCHUNK, zrow, 0)

        def zchunk(r, _):
            pltpu.sync_copy(zbuf, acc.at[pl.ds(row0 + r * ZCHUNK, ZCHUNK)])
            return 0

        lax.fori_loop(0, ROWS_TILE // ZCHUNK, zchunk, 0)
        plsc.subcore_barrier()

        def gwait(buf, sem):
            pltpu.make_async_copy(xs.at[src_v.at[0]], buf, sem).wait()

        def swait(buf, sem):
            pltpu.make_async_copy(buf, acc.at[dst_v.at[0]], sem).wait()

        for h in range(NH):
            pltpu.sync_copy(src_hbm.at[s, h], src_v)
            pltpu.sync_copy(dst_hbm.at[s, h], dst_v)
            pltpu.async_copy(xs.at[src_v.at[0]], rows_a, sem_ga)

            def pair(jj, _):
                j = 2 * jj
                gwait(rows_a, sem_ga)
                pltpu.async_copy(rows_a, acc.at[dst_v.at[j]], sem_sa, add=True)

                @pl.when(jj > 0)
                def _():
                    swait(rows_b, sem_sb)

                pltpu.async_copy(xs.at[src_v.at[j + 1]], rows_b, sem_gb)
                gwait(rows_b, sem_gb)
                pltpu.async_copy(rows_b, acc.at[dst_v.at[j + 1]], sem_sb, add=True)
                swait(rows_a, sem_sa)

                @pl.when(jj + 1 < CH_PER_HALF // 2)
                def _():
                    pltpu.async_copy(xs.at[src_v.at[j + 2]], rows_a, sem_ga)

                return 0

            lax.fori_loop(0, CH_PER_HALF // 2, pair, 0)
            swait(rows_b, sem_sb)

        plsc.subcore_barrier()

        pltpu.sync_copy(acc.at[pl.ds(row0, ROWS_TILE)],
                        out_hbm.at[c, pl.ds(row0, ROWS_TILE)])

    return k(xh, src2, dst2)


def _combine(x, p):
    def body(x_ref, p_ref, o_ref):
        w = jnp.float32(WEIGHT)
        o_ref[:, :HALF] = x_ref[:, :HALF] + w * p_ref[0]
        o_ref[:, HALF:] = x_ref[:, HALF:] + w * p_ref[1]

    return pl.pallas_call(
        body,
        out_shape=jax.ShapeDtypeStruct((N_NODES, D_FEAT), jnp.float32),
        grid=(10,),
        in_specs=[
            pl.BlockSpec((1000, D_FEAT), lambda i: (i, 0)),
            pl.BlockSpec((NC, 1000, HALF), lambda i: (0, i, 0)),
        ],
        out_specs=pl.BlockSpec((1000, D_FEAT), lambda i: (i, 0)),
    )(x, p)


@jax.jit
def kernel(x, edge_index):
    src = edge_index[0]
    dst = edge_index[1]
    xh = jnp.concatenate([x[:, :HALF], x[:, HALF:]], axis=0)
    srcp = jnp.pad(src, (0, E_PAD - N_EDGES))
    dstp = jnp.pad(dst, (0, E_PAD - N_EDGES), constant_values=N_NODES)
    src2 = srcp.reshape(NS, NH, CH_PER_HALF, CHUNK)
    dst2 = dstp.reshape(NS, NH, CH_PER_HALF, CHUNK)
    p = _sc_agg(xh, src2, dst2)
    return _combine(x, p)


# speedup vs baseline: 1.7815x; 1.0912x over previous
import functools

import jax
import jax.numpy as jnp
from jax import lax
from jax.experimental import pallas as pl
from jax.experimental.pallas import tpu as pltpu
from jax.experimental.pallas import tpu_sc as plsc

N_NODES = 10000
D_FEAT = 128
HALF = 64
N_EDGES = 320000
NC = 2
NS = 16
CHUNK = 128
NH = 4
CH_PER_HALF = 40
CH_PER_TILE = NH * CH_PER_HALF
E_TILE = CHUNK * CH_PER_TILE
E_PAD = E_TILE * NS
ROWS_PAD = 10240
ROWS_TILE = ROWS_PAD // NS
X_TILE = N_NODES // NS
ZCHUNK = 16
WEIGHT = 0.1


def _sc_agg(xh, src2, dst2):
    mesh = plsc.VectorSubcoreMesh(core_axis_name="c", subcore_axis_name="s")

    @functools.partial(
        pl.kernel,
        out_type=jax.ShapeDtypeStruct((NC, ROWS_PAD, HALF), jnp.float32),
        mesh=mesh,
        scratch_types=[
            pltpu.VMEM((CH_PER_HALF, CHUNK), jnp.int32),
            pltpu.VMEM((CH_PER_HALF, CHUNK), jnp.int32),
            pltpu.VMEM((CHUNK, HALF), jnp.float32),
            pltpu.VMEM((CHUNK, HALF), jnp.float32),
            pltpu.VMEM((CHUNK, HALF), jnp.float32),
            pltpu.VMEM((CHUNK, HALF), jnp.float32),
            pltpu.VMEM((ZCHUNK, HALF), jnp.float32),
            pltpu.VMEM_SHARED((N_NODES, HALF), jnp.float32),
            pltpu.VMEM_SHARED((ROWS_PAD, HALF), jnp.float32),
            [pltpu.SemaphoreType.DMA] * 4,
            [pltpu.SemaphoreType.DMA] * 4,
        ],
        compiler_params=pltpu.CompilerParams(use_tc_tiling_on_sc=False),
    )
    def k(xh_hbm, src_hbm, dst_hbm, out_hbm, src_v, dst_v, rows_a, rows_b,
          rows_c, rows_d, zbuf, xs, acc, sem_g, sem_s):
        c = lax.axis_index("c")
        s = lax.axis_index("s")
        row0 = s * ROWS_TILE

        x0 = s * X_TILE
        pltpu.sync_copy(xh_hbm.at[pl.ds(c * N_NODES + x0, X_TILE)],
                        xs.at[pl.ds(x0, X_TILE)])

        zero = jnp.zeros((16,), jnp.float32)

        def zrow(i, _):
            for j in range(HALF // 16):
                zbuf[i, pl.ds(j * 16, 16)] = zero
            return 0

        lax.fori_loop(0, ZCHUNK, zrow, 0)

        def zchunk(r, _):
            pltpu.sync_copy(zbuf, acc.at[pl.ds(row0 + r * ZCHUNK, ZCHUNK)])
            return 0

        lax.fori_loop(0, ROWS_TILE // ZCHUNK, zchunk, 0)
        plsc.subcore_barrier()

        bufs = (rows_a, rows_b, rows_c, rows_d)
        NQ = CH_PER_HALF // 4

        def gissue(j, b):
            pltpu.async_copy(xs.at[src_v.at[j]], bufs[b], sem_g[b])

        def gwait(b):
            pltpu.make_async_copy(xs.at[src_v.at[0]], bufs[b], sem_g[b]).wait()

        def sissue(j, b):
            pltpu.async_copy(bufs[b], acc.at[dst_v.at[j]], sem_s[b], add=True)

        def swait(b):
            pltpu.make_async_copy(bufs[b], acc.at[dst_v.at[0]], sem_s[b]).wait()

        for h in range(NH):
            pltpu.sync_copy(src_hbm.at[s, h], src_v)
            pltpu.sync_copy(dst_hbm.at[s, h], dst_v)
            gissue(0, 0)
            gissue(1, 1)

            def quad(jj, _):
                j = 4 * jj
                gwait(0)
                sissue(j, 0)

                @pl.when(jj > 0)
                def _():
                    swait(2)

                gissue(j + 2, 2)
                gwait(1)
                sissue(j + 1, 1)

                @pl.when(jj > 0)
                def _():
                    swait(3)

                gissue(j + 3, 3)
                gwait(2)
                sissue(j + 2, 2)
                swait(0)

                @pl.when(jj + 1 < NQ)
                def _():
                    gissue(j + 4, 0)

                gwait(3)
                sissue(j + 3, 3)
                swait(1)

                @pl.when(jj + 1 < NQ)
                def _():
                    gissue(j + 5, 1)

                return 0

            lax.fori_loop(0, NQ, quad, 0)
            swait(2)
            swait(3)

        plsc.subcore_barrier()

        pltpu.sync_copy(acc.at[pl.ds(row0, ROWS_TILE)],
                        out_hbm.at[c, pl.ds(row0, ROWS_TILE)])

    return k(xh, src2, dst2)


def _combine(x, p):
    def body(x_ref, p_ref, o_ref):
        w = jnp.float32(WEIGHT)
        o_ref[:, :HALF] = x_ref[:, :HALF] + w * p_ref[0]
        o_ref[:, HALF:] = x_ref[:, HALF:] + w * p_ref[1]

    return pl.pallas_call(
        body,
        out_shape=jax.ShapeDtypeStruct((N_NODES, D_FEAT), jnp.float32),
        grid=(10,),
        in_specs=[
            pl.BlockSpec((1000, D_FEAT), lambda i: (i, 0)),
            pl.BlockSpec((NC, 1000, HALF), lambda i: (0, i, 0)),
        ],
        out_specs=pl.BlockSpec((1000, D_FEAT), lambda i: (i, 0)),
    )(x, p)


@jax.jit
def kernel(x, edge_index):
    src = edge_index[0]
    dst = edge_index[1]
    xh = jnp.concatenate([x[:, :HALF], x[:, HALF:]], axis=0)
    srcp = jnp.pad(src, (0, E_PAD - N_EDGES))
    dstp = jnp.pad(dst, (0, E_PAD - N_EDGES), constant_values=N_NODES)
    src2 = srcp.reshape(NS, NH, CH_PER_HALF, CHUNK)
    dst2 = dstp.reshape(NS, NH, CH_PER_HALF, CHUNK)
    p = _sc_agg(xh, src2, dst2)
    return _combine(x, p)


# speedup vs baseline: 2.0141x; 1.1306x over previous
import functools

import jax
import jax.numpy as jnp
from jax import lax
from jax.experimental import pallas as pl
from jax.experimental.pallas import tpu as pltpu
from jax.experimental.pallas import tpu_sc as plsc

N_NODES = 10000
D_FEAT = 128
HALF = 64
N_EDGES = 320000
NC = 2
NS = 16
CHUNK = 128
NH = 4
CH_PER_HALF = 40
CH_PER_TILE = NH * CH_PER_HALF
E_TILE = CHUNK * CH_PER_TILE
E_PAD = E_TILE * NS
ROWS_PAD = 10240
ROWS_TILE = ROWS_PAD // NS
X_TILE = N_NODES // NS
ZCHUNK = 16
WEIGHT = 0.1


def _sc_agg(xh, src2, dst2):
    mesh = plsc.VectorSubcoreMesh(core_axis_name="c", subcore_axis_name="s")

    @functools.partial(
        pl.kernel,
        out_type=jax.ShapeDtypeStruct((NC, ROWS_PAD, HALF), jnp.float32),
        mesh=mesh,
        scratch_types=[
            pltpu.VMEM((CH_PER_HALF, CHUNK), jnp.int32),
            pltpu.VMEM((CH_PER_HALF, CHUNK), jnp.int32),
            pltpu.VMEM((CHUNK, HALF), jnp.float32),
            pltpu.VMEM((CHUNK, HALF), jnp.float32),
            pltpu.VMEM((CHUNK, HALF), jnp.float32),
            pltpu.VMEM((CHUNK, HALF), jnp.float32),
            pltpu.VMEM_SHARED((N_NODES, HALF), jnp.float32),
            pltpu.VMEM_SHARED((ROWS_PAD, HALF), jnp.float32),
            [pltpu.SemaphoreType.DMA] * 4,
            [pltpu.SemaphoreType.DMA] * 4,
        ],
        compiler_params=pltpu.CompilerParams(use_tc_tiling_on_sc=False),
    )
    def k(xh_hbm, src_hbm, dst_hbm, out_hbm, src_v, dst_v, rows_a, rows_b,
          rows_c, rows_d, xs, acc, sem_g, sem_s):
        c = lax.axis_index("c")
        s = lax.axis_index("s")
        row0 = s * ROWS_TILE

        x0 = s * X_TILE
        pltpu.sync_copy(xh_hbm.at[pl.ds(x0, X_TILE), pl.ds(c * HALF, HALF)],
                        xs.at[pl.ds(x0, X_TILE)])

        zero = jnp.zeros((16,), jnp.float32)

        def zrow(i, _):
            for j in range(HALF // 16):
                rows_a[i, pl.ds(j * 16, 16)] = zero
            return 0

        lax.fori_loop(0, CHUNK, zrow, 0)
        nz = ROWS_TILE // CHUNK
        for r in range(nz):
            pltpu.async_copy(rows_a, acc.at[pl.ds(row0 + r * CHUNK, CHUNK)],
                             sem_g[r % 4])
        for r in range(nz):
            pltpu.make_async_copy(
                rows_a, acc.at[pl.ds(row0 + r * CHUNK, CHUNK)],
                sem_g[r % 4]).wait()
        plsc.subcore_barrier()

        bufs = (rows_a, rows_b, rows_c, rows_d)
        NQ = CH_PER_HALF // 4

        def gissue(j, b):
            pltpu.async_copy(xs.at[src_v.at[j]], bufs[b], sem_g[b])

        def gwait(b):
            pltpu.make_async_copy(xs.at[src_v.at[0]], bufs[b], sem_g[b]).wait()

        def sissue(j, b):
            pltpu.async_copy(bufs[b], acc.at[dst_v.at[j]], sem_s[b], add=True)

        def swait(b):
            pltpu.make_async_copy(bufs[b], acc.at[dst_v.at[0]], sem_s[b]).wait()

        for h in range(NH):
            pltpu.sync_copy(src_hbm.at[s, h], src_v)
            pltpu.sync_copy(dst_hbm.at[s, h], dst_v)
            gissue(0, 0)
            gissue(1, 1)

            def quad(jj, _):
                j = 4 * jj
                gwait(0)
                sissue(j, 0)

                @pl.when(jj > 0)
                def _():
                    swait(2)

                gissue(j + 2, 2)
                gwait(1)
                sissue(j + 1, 1)

                @pl.when(jj > 0)
                def _():
                    swait(3)

                gissue(j + 3, 3)
                gwait(2)
                sissue(j + 2, 2)
                swait(0)

                @pl.when(jj + 1 < NQ)
                def _():
                    gissue(j + 4, 0)

                gwait(3)
                sissue(j + 3, 3)
                swait(1)

                @pl.when(jj + 1 < NQ)
                def _():
                    gissue(j + 5, 1)

                return 0

            lax.fori_loop(0, NQ, quad, 0)
            swait(2)
            swait(3)

        plsc.subcore_barrier()

        pltpu.sync_copy(acc.at[pl.ds(row0, ROWS_TILE)],
                        out_hbm.at[c, pl.ds(row0, ROWS_TILE)])

    return k(xh, src2, dst2)


def _combine(x, p):
    def body(x_ref, p_ref, o_ref):
        w = jnp.float32(WEIGHT)
        o_ref[:, :HALF] = x_ref[:, :HALF] + w * p_ref[0]
        o_ref[:, HALF:] = x_ref[:, HALF:] + w * p_ref[1]

    return pl.pallas_call(
        body,
        out_shape=jax.ShapeDtypeStruct((N_NODES, D_FEAT), jnp.float32),
        grid=(10,),
        in_specs=[
            pl.BlockSpec((1000, D_FEAT), lambda i: (i, 0)),
            pl.BlockSpec((NC, 1000, HALF), lambda i: (0, i, 0)),
        ],
        out_specs=pl.BlockSpec((1000, D_FEAT), lambda i: (i, 0)),
    )(x, p)


@jax.jit
def kernel(x, edge_index):
    src = edge_index[0]
    dst = edge_index[1]
    srcp = jnp.pad(src, (0, E_PAD - N_EDGES))
    dstp = jnp.pad(dst, (0, E_PAD - N_EDGES), constant_values=N_NODES)
    src2 = srcp.reshape(NS, NH, CH_PER_HALF, CHUNK)
    dst2 = dstp.reshape(NS, NH, CH_PER_HALF, CHUNK)
    p = _sc_agg(x, src2, dst2)
    return _combine(x, p)
